# baseline (speedup 1.0000x reference)
import functools

import jax
import jax.numpy as jnp
from jax import lax
from jax.experimental import pallas as pl
from jax.experimental.pallas import tpu as pltpu
from jax.experimental.pallas import tpu_sc as plsc

NC = 2
NS = 16
NW = NC * NS

B = 16384
E = 16
N_DNS = 10
N_DSEQ = 50
N_SNS = 26
N_SSEQ = 20
G = 5

C = 16
B_PER_W = B // NW
N_CHUNKS = B_PER_W // C

OFF_DNS = 0
OFF_DSEQ = C * N_DNS
OFF_SNS = OFF_DSEQ + C * N_DSEQ
OFF_SSEQ = OFF_SNS + C * N_SNS
IDS_PER_CHUNK = OFF_SSEQ + C * N_SSEQ


def _streams():
  out = []
  for dense, off, n in ((True, OFF_DNS, C * N_DNS),
                        (True, OFF_DSEQ, C * N_DSEQ),
                        (False, OFF_SNS, C * N_SNS),
                        (False, OFF_SSEQ, C * N_SSEQ)):
    p = 0
    while p < n:
      sz = min(128, n - p)
      out.append((dense, off + p, sz))
      p += sz
  return out


STREAMS = _streams()

TW = 4096
V = 1000000
NTB = -(-V // TW)
VP = NTB * TW


def _sc_body(dense_table, sparse_table, ids_cat,
             out0, out1, out2, out3, out4,
             idx0, idx1, buf0, buf1,
             a00, a01, a02, a03, a04,
             a10, a11, a12, a13, a14,
             gsem0, gsem1, isem, wsem0, wsem1):
  wid = lax.axis_index("s") * NC + lax.axis_index("c")
  base_chunk = wid * N_CHUNKS
  zeros = jnp.zeros((16,), jnp.float32)
  aset0 = (a00, a01, a02, a03, a04)
  aset1 = (a10, a11, a12, a13, a14)
  outs = (out0, out1, out2, out3, out4)

  def stage(c, idx, sem):
    off = pl.multiple_of((base_chunk + c) * IDS_PER_CHUNK, 8)
    return pltpu.async_copy(ids_cat.at[pl.ds(off, IDS_PER_CHUNK)], idx, sem)

  def wait_ids(idx):
    pltpu.make_async_copy(ids_cat.at[pl.ds(0, IDS_PER_CHUNK)], idx, isem).wait()

  def permute(idx):
    def pv(t, _):
      v = idx[pl.ds(t * 16, 16)]
      g = ((v & jnp.int32(~(TW - 1)))
           | ((v & jnp.int32(511)) << 3)
           | ((v >> 9) & jnp.int32(7)))
      idx[pl.ds(t * 16, 16)] = g
      return 0
    lax.fori_loop(0, IDS_PER_CHUNK // 16, pv, 0)

  def fire(idx, buf, gsem):
    for dense, off, sz in STREAMS:
      table = dense_table if dense else sparse_table
      pltpu.async_copy(table.at[idx.at[pl.ds(off, sz)]],
                       buf.at[pl.ds(off, sz), :], gsem)

  def drain(gsem):
    for _, _, sz in STREAMS:
      pltpu.make_async_copy(dense_table.at[idx0.at[pl.ds(0, sz)]],
                            buf0.at[pl.ds(0, sz), :], gsem).wait()

  def drain_writes(aset, wsem):
    for k in range(G):
      pltpu.make_async_copy(aset[k], outs[k].at[pl.ds(0, C), :], wsem).wait()

  def process(c, buf, aset, wsem):
    a0, a1, a2, a3, a4 = aset

    def row_body(i, _):
      db = OFF_DSEQ + i * N_DSEQ
      acc = buf[db]
      for t in range(1, N_DSEQ):
        acc = acc + buf[db + t]
      dmean = acc * (1.0 / N_DSEQ)
      sb = OFF_SSEQ + i * N_SSEQ
      acc2 = buf[sb]
      for t in range(1, N_SSEQ):
        acc2 = acc2 + buf[sb + t]
      smean = acc2 * (1.0 / N_SSEQ)

      dn = OFF_DNS + i * N_DNS
      sn = OFF_SNS + i * N_SNS
      for j in range(8):
        a0[i, pl.ds(16 * j, 16)] = buf[dn + j]
      a1[i, pl.ds(0, 16)] = buf[dn + 8]
      a1[i, pl.ds(16, 16)] = buf[dn + 9]
      a1[i, pl.ds(32, 16)] = dmean
      for j in range(5):
        a1[i, pl.ds(48 + 16 * j, 16)] = buf[sn + j]
      for j in range(8):
        a2[i, pl.ds(16 * j, 16)] = buf[sn + 5 + j]
      for j in range(8):
        a3[i, pl.ds(16 * j, 16)] = buf[sn + 13 + j]
      for j in range(5):
        a4[i, pl.ds(16 * j, 16)] = buf[sn + 21 + j]
      a4[i, pl.ds(80, 16)] = smean
      a4[i, pl.ds(96, 16)] = zeros
      a4[i, pl.ds(112, 16)] = zeros
      return 0
    lax.fori_loop(0, C, row_body, 0)

    off = pl.multiple_of(wid * B_PER_W + c * C, C)
    for k in range(G):
      pltpu.async_copy(aset[k], outs[k].at[pl.ds(off, C), :], wsem)

  pltpu.sync_copy(
      ids_cat.at[pl.ds(pl.multiple_of(base_chunk * IDS_PER_CHUNK, 8),
                       IDS_PER_CHUNK)], idx0)
  permute(idx0)
  fire(idx0, buf0, gsem0)
  stage(1, idx1, isem)

  def pair_body(h, _):
    ca = 2 * h

    wait_ids(idx1)
    permute(idx1)
    fire(idx1, buf1, gsem1)
    drain(gsem0)

    @pl.when(h < (N_CHUNKS // 2) - 1)
    def _():
      stage(ca + 2, idx0, isem)

    @pl.when(h > 0)
    def _():
      drain_writes(aset0, wsem0)
    process(ca, buf0, aset0, wsem0)

    @pl.when(h < (N_CHUNKS // 2) - 1)
    def _():
      wait_ids(idx0)
      permute(idx0)
      fire(idx0, buf0, gsem0)
    drain(gsem1)

    @pl.when(h < (N_CHUNKS // 2) - 1)
    def _():
      stage(ca + 3, idx1, isem)

    @pl.when(h > 0)
    def _():
      drain_writes(aset1, wsem1)
    process(ca + 1, buf1, aset1, wsem1)
    return 0

  lax.fori_loop(0, N_CHUNKS // 2, pair_body, 0)
  drain_writes(aset0, wsem0)
  drain_writes(aset1, wsem1)


_sc_gather = functools.partial(
    pl.kernel,
    out_type=tuple(jax.ShapeDtypeStruct((B, 128), jnp.float32) for _ in range(G)),
    mesh=plsc.VectorSubcoreMesh(core_axis_name="c", subcore_axis_name="s"),
    compiler_params=pltpu.CompilerParams(use_tc_tiling_on_sc=False),
    scratch_types=[
        pltpu.VMEM((IDS_PER_CHUNK,), jnp.int32),
        pltpu.VMEM((IDS_PER_CHUNK,), jnp.int32),
        pltpu.VMEM((IDS_PER_CHUNK, E), jnp.float32),
        pltpu.VMEM((IDS_PER_CHUNK, E), jnp.float32),
    ]
    + [pltpu.VMEM((C, 128), jnp.float32) for _ in range(2 * G)]
    + [pltpu.SemaphoreType.DMA] * 5,
)(_sc_body)


def _tr_body(in_ref, out_ref):
  x = in_ref[...]
  xs = jnp.concatenate([x[:, 512 * k:512 * (k + 1)] for k in range(8)], axis=0)
  eye = jnp.eye(128, dtype=jnp.float32)
  out_ref[...] = lax.dot_general(xs, eye, (((0,), (0,)), ((), ())),
                                 preferred_element_type=jnp.float32)


def _tc_transpose(table_t):
  out = pl.pallas_call(
      _tr_body,
      grid=(NTB,),
      in_specs=[pl.BlockSpec((E, TW), lambda i: (0, i))],
      out_specs=pl.BlockSpec((TW // 8, 128), lambda i: (i, 0)),
      out_shape=jax.ShapeDtypeStruct((VP * E // 128, 128), jnp.float32),
  )(table_t)
  return out.reshape(VP, E)


BM = 512


def _tc_body(x0_ref, x1_ref, x2_ref, x3_ref, x4_ref, w_ref, b_ref, out_ref):
  w = w_ref[...]
  acc = jnp.dot(x0_ref[...], w[0:128], preferred_element_type=jnp.float32)
  acc += jnp.dot(x1_ref[...], w[128:256], preferred_element_type=jnp.float32)
  acc += jnp.dot(x2_ref[...], w[256:384], preferred_element_type=jnp.float32)
  acc += jnp.dot(x3_ref[...], w[384:512], preferred_element_type=jnp.float32)
  acc += jnp.dot(x4_ref[...], w[512:640], preferred_element_type=jnp.float32)
  acc += b_ref[0:1, :]
  out_ref[...] = jnp.maximum(acc, 0.0)


def _tc_matmul(xs, wpad, b8):
  grid = (B // BM,)
  return pl.pallas_call(
      _tc_body,
      grid=grid,
      in_specs=[pl.BlockSpec((BM, 128), lambda i: (i, 0)) for _ in range(G)]
      + [
          pl.BlockSpec((G * 128, 128), lambda i: (0, 0)),
          pl.BlockSpec((8, 128), lambda i: (0, 0)),
      ],
      out_specs=pl.BlockSpec((BM, 128), lambda i: (i, 0)),
      out_shape=jax.ShapeDtypeStruct((B, 128), jnp.float32),
  )(*xs, wpad, b8)


def kernel(dense_scalar_ids, dense_seq_ids, sparse_scalar_ids, sparse_seq_ids,
           dense_table, sparse_table, W, b):
  nch = B // C
  ids_cat = jnp.concatenate(
      [dense_scalar_ids.reshape(nch, C * N_DNS),
       dense_seq_ids.reshape(nch, C * N_DSEQ),
       sparse_scalar_ids.reshape(nch, C * N_SNS),
       sparse_seq_ids.reshape(nch, C * N_SSEQ)], axis=1).reshape(-1)
  dt = _tc_transpose(dense_table.T)
  st = _tc_transpose(sparse_table.T)
  xs = _sc_gather(dt, st, ids_cat)
  wpad = jnp.concatenate([W, jnp.zeros((G * 128 - 608, 128), W.dtype)], axis=0)
  b8 = jnp.broadcast_to(b, (8, 128))
  return _tc_matmul(xs, wpad, b8)


# speedup vs baseline: 1.1191x; 1.1191x over previous
import functools

import jax
import jax.numpy as jnp
from jax import lax
from jax.experimental import pallas as pl
from jax.experimental.pallas import tpu as pltpu
from jax.experimental.pallas import tpu_sc as plsc

NC = 2
NS = 16
NW = NC * NS

B = 16384
E = 16
N_DNS = 10
N_DSEQ = 50
N_SNS = 26
N_SSEQ = 20

C = 16
B_PER_W = B // NW
N_CHUNKS = B_PER_W // C

TW = 4096
V = 1000000
NTB = -(-V // TW)
VP = NTB * TW


def _streams(segs):
  out = []
  for off, n in segs:
    p = 0
    while p < n:
      sz = min(128, n - p)
      out.append((off + p, sz))
      p += sz
  return out


def _make_sc_kernel(n_scalar, n_seq, n_groups, assemble):
  off_seq = C * n_scalar
  ipc = C * (n_scalar + n_seq)
  streams = _streams([(0, C * n_scalar), (off_seq, C * n_seq)])

  def body(table, ids_cat, *rest):
    outs = rest[:n_groups]
    idx0, idx1, buf0, buf1 = rest[n_groups:n_groups + 4]
    a = rest[n_groups + 4:n_groups + 4 + 2 * n_groups]
    aset0, aset1 = a[:n_groups], a[n_groups:]
    gsem0, gsem1, isem, wsem0, wsem1 = rest[n_groups + 4 + 2 * n_groups:]
    wid = lax.axis_index("s") * NC + lax.axis_index("c")
    base_chunk = wid * N_CHUNKS
    zeros = jnp.zeros((16,), jnp.float32)

    def stage(c, idx):
      off = pl.multiple_of((base_chunk + c) * ipc, 8)
      pltpu.async_copy(ids_cat.at[pl.ds(off, ipc)], idx, isem)

    def wait_ids(idx):
      pltpu.make_async_copy(ids_cat.at[pl.ds(0, ipc)], idx, isem).wait()

    def permute(idx):
      def pv(t, _):
        v = idx[pl.ds(t * 16, 16)]
        g = ((v & jnp.int32(~(TW - 1)))
             | ((v & jnp.int32(511)) << 3)
             | ((v >> 9) & jnp.int32(7)))
        idx[pl.ds(t * 16, 16)] = g
        return 0
      lax.fori_loop(0, ipc // 16, pv, 0)

    def fire(idx, buf, gsem):
      for off, sz in streams:
        pltpu.async_copy(table.at[idx.at[pl.ds(off, sz)]],
                         buf.at[pl.ds(off, sz), :], gsem)

    def drain(gsem):
      for _, sz in streams:
        pltpu.make_async_copy(table.at[idx0.at[pl.ds(0, sz)]],
                              buf0.at[pl.ds(0, sz), :], gsem).wait()

    def drain_writes(aset, wsem):
      for k in range(n_groups):
        pltpu.make_async_copy(aset[k], outs[k].at[pl.ds(0, C), :], wsem).wait()

    def process(c, buf, aset, wsem):
      def row_body(i, _):
        sb = off_seq + i * n_seq
        acc = buf[sb]
        for t in range(1, n_seq):
          acc = acc + buf[sb + t]
        mean = acc * (1.0 / n_seq)
        assemble(buf, i, mean, aset, zeros)
        return 0
      lax.fori_loop(0, C, row_body, 0)
      off = pl.multiple_of(wid * B_PER_W + c * C, C)
      for k in range(n_groups):
        pltpu.async_copy(aset[k], outs[k].at[pl.ds(off, C), :], wsem)

    pltpu.sync_copy(
        ids_cat.at[pl.ds(pl.multiple_of(base_chunk * ipc, 8), ipc)], idx0)
    permute(idx0)
    fire(idx0, buf0, gsem0)
    stage(1, idx1)

    def pair_body(h, _):
      ca = 2 * h
      wait_ids(idx1)
      permute(idx1)
      fire(idx1, buf1, gsem1)
      drain(gsem0)

      @pl.when(h < (N_CHUNKS // 2) - 1)
      def _():
        stage(ca + 2, idx0)

      @pl.when(h > 0)
      def _():
        drain_writes(aset0, wsem0)
      process(ca, buf0, aset0, wsem0)

      @pl.when(h < (N_CHUNKS // 2) - 1)
      def _():
        wait_ids(idx0)
        permute(idx0)
        fire(idx0, buf0, gsem0)
      drain(gsem1)

      @pl.when(h < (N_CHUNKS // 2) - 1)
      def _():
        stage(ca + 3, idx1)

      @pl.when(h > 0)
      def _():
        drain_writes(aset1, wsem1)
      process(ca + 1, buf1, aset1, wsem1)
      return 0

    lax.fori_loop(0, N_CHUNKS // 2, pair_body, 0)
    drain_writes(aset0, wsem0)
    drain_writes(aset1, wsem1)

  return functools.partial(
      pl.kernel,
      out_type=tuple(jax.ShapeDtypeStruct((B, 128), jnp.float32)
                     for _ in range(n_groups)),
      mesh=plsc.VectorSubcoreMesh(core_axis_name="c", subcore_axis_name="s"),
      compiler_params=pltpu.CompilerParams(use_tc_tiling_on_sc=False),
      scratch_types=[
          pltpu.VMEM((ipc,), jnp.int32),
          pltpu.VMEM((ipc,), jnp.int32),
          pltpu.VMEM((ipc, E), jnp.float32),
          pltpu.VMEM((ipc, E), jnp.float32),
      ]
      + [pltpu.VMEM((C, 128), jnp.float32) for _ in range(2 * n_groups)]
      + [pltpu.SemaphoreType.DMA] * 5,
  )(body)


def _assemble_dense(buf, i, mean, aset, zeros):
  a0, a1 = aset
  dn = i * N_DNS
  for j in range(8):
    a0[i, pl.ds(16 * j, 16)] = buf[dn + j]
  a1[i, pl.ds(0, 16)] = buf[dn + 8]
  a1[i, pl.ds(16, 16)] = buf[dn + 9]
  a1[i, pl.ds(32, 16)] = mean
  for j in range(5):
    a1[i, pl.ds(48 + 16 * j, 16)] = zeros


def _assemble_sparse(buf, i, mean, aset, zeros):
  a0, a1, a2, a3 = aset
  sn = i * N_SNS
  for j in range(8):
    a0[i, pl.ds(16 * j, 16)] = buf[sn + j]
  for j in range(8):
    a1[i, pl.ds(16 * j, 16)] = buf[sn + 8 + j]
  for j in range(8):
    a2[i, pl.ds(16 * j, 16)] = buf[sn + 16 + j]
  a3[i, pl.ds(0, 16)] = buf[sn + 24]
  a3[i, pl.ds(16, 16)] = buf[sn + 25]
  a3[i, pl.ds(32, 16)] = mean
  for j in range(5):
    a3[i, pl.ds(48 + 16 * j, 16)] = zeros


_sc_dense = _make_sc_kernel(N_DNS, N_DSEQ, 2, _assemble_dense)
_sc_sparse = _make_sc_kernel(N_SNS, N_SSEQ, 4, _assemble_sparse)


def _tr_body(in_ref, out_ref):
  x = in_ref[...]
  xs = jnp.concatenate([x[:, 512 * k:512 * (k + 1)] for k in range(8)], axis=0)
  out_ref[...] = xs.T


def _tc_transpose(table_t):
  out = pl.pallas_call(
      _tr_body,
      grid=(NTB,),
      in_specs=[pl.BlockSpec((E, TW), lambda i: (0, i))],
      out_specs=pl.BlockSpec((TW // 8, 128), lambda i: (i, 0)),
      out_shape=jax.ShapeDtypeStruct((VP * E // 128, 128), jnp.float32),
  )(table_t)
  return out.reshape(VP, E)


BM = 512
NG = 6


def _tc_body(x0_ref, x1_ref, x2_ref, x3_ref, x4_ref, x5_ref,
             w_ref, b_ref, out_ref):
  w = w_ref[...]
  xs = (x0_ref, x1_ref, x2_ref, x3_ref, x4_ref, x5_ref)
  acc = b_ref[0:1, :].astype(jnp.float32) + jnp.zeros((BM, 128), jnp.float32)
  for k in range(NG):
    acc += jnp.dot(xs[k][...], w[128 * k:128 * (k + 1)],
                   preferred_element_type=jnp.float32)
  out_ref[...] = jnp.maximum(acc, 0.0)


def _tc_matmul(xs, wpad, b8):
  grid = (B // BM,)
  return pl.pallas_call(
      _tc_body,
      grid=grid,
      in_specs=[pl.BlockSpec((BM, 128), lambda i: (i, 0)) for _ in range(NG)]
      + [
          pl.BlockSpec((NG * 128, 128), lambda i: (0, 0)),
          pl.BlockSpec((8, 128), lambda i: (0, 0)),
      ],
      out_specs=pl.BlockSpec((BM, 128), lambda i: (i, 0)),
      out_shape=jax.ShapeDtypeStruct((B, 128), jnp.float32),
  )(*xs, wpad, b8)


def kernel(dense_scalar_ids, dense_seq_ids, sparse_scalar_ids, sparse_seq_ids,
           dense_table, sparse_table, W, b):
  nch = B // C
  dense_cat = jnp.concatenate(
      [dense_scalar_ids.reshape(nch, C * N_DNS),
       dense_seq_ids.reshape(nch, C * N_DSEQ)], axis=1).reshape(-1)
  sparse_cat = jnp.concatenate(
      [sparse_scalar_ids.reshape(nch, C * N_SNS),
       sparse_seq_ids.reshape(nch, C * N_SSEQ)], axis=1).reshape(-1)
  dt = _tc_transpose(dense_table.T)
  xd = _sc_dense(dt, dense_cat)
  st = _tc_transpose(sparse_table.T)
  xsp = _sc_sparse(st, sparse_cat)
  zpad = jnp.zeros((80, 128), W.dtype)
  wpad = jnp.concatenate([W[0:176], zpad, W[176:608], zpad], axis=0)
  b8 = jnp.broadcast_to(b, (8, 128))
  return _tc_matmul(list(xd) + list(xsp), wpad, b8)
